# baseline (speedup 1.0000x reference)
import jax
import jax.numpy as jnp
from jax.experimental import pallas as pl
from jax.experimental.pallas import tpu as pltpu

INC = 32
OUTC = 64
INN = 8192
OUTN = 64
MAXD = 8
NB = 128
USED = OUTN * MAXD
BLK = 64


def _fgl_kernel(x_ref, nf_ref, ft_ref, bias_ref, out_ref):
    xb = x_ref[...]
    nf = nf_ref[...]
    h = xb * nf[None, :, :]
    t = jnp.maximum(h, pltpu.roll(h, shift=USED - 1, axis=2))
    t = jnp.maximum(t, pltpu.roll(t, shift=USED - 2, axis=2))
    t = jnp.maximum(t, pltpu.roll(t, shift=USED - 4, axis=2))
    jj = jax.lax.broadcasted_iota(jnp.int32, (USED, OUTN), 0)
    oo = jax.lax.broadcasted_iota(jnp.int32, (USED, OUTN), 1)
    sel = (jj == oo * MAXD).astype(jnp.float32)
    m = jax.lax.dot_general(
        t, sel,
        dimension_numbers=(((2,), (0,)), ((), ())),
        preferred_element_type=jnp.float32,
    )
    ft = ft_ref[...]
    out = jax.lax.dot_general(
        m, ft,
        dimension_numbers=(((1,), (0,)), ((), ())),
        preferred_element_type=jnp.float32,
    )
    out = out.transpose(0, 2, 1)
    out_ref[...] = out + bias_ref[...][None, :, :]


def kernel(x, nf_weight, ft_weight, bias):
    grid = (NB // BLK,)
    return pl.pallas_call(
        _fgl_kernel,
        grid=grid,
        in_specs=[
            pl.BlockSpec((BLK, INC, USED), lambda j: (j, 0, 0)),
            pl.BlockSpec((INC, USED), lambda j: (0, 0)),
            pl.BlockSpec((INC, OUTC), lambda j: (0, 0)),
            pl.BlockSpec((OUTC, OUTN), lambda j: (0, 0)),
        ],
        out_specs=pl.BlockSpec((BLK, OUTC, OUTN), lambda j: (j, 0, 0)),
        out_shape=jax.ShapeDtypeStruct((NB, OUTC, OUTN), jnp.float32),
    )(x, nf_weight, ft_weight, bias)


# speedup vs baseline: 1.0508x; 1.0508x over previous
import jax
import jax.numpy as jnp
from jax.experimental import pallas as pl
from jax.experimental.pallas import tpu as pltpu

INC = 32
OUTC = 64
INN = 8192
OUTN = 64
MAXD = 8
NB = 128
USED = OUTN * MAXD
BLK = 32
HALF = BLK // 2


def _grouped_max(h):
    t = jnp.maximum(h, pltpu.roll(h, shift=USED - 1, axis=2))
    t = jnp.maximum(t, pltpu.roll(t, shift=USED - 2, axis=2))
    t = jnp.maximum(t, pltpu.roll(t, shift=USED - 4, axis=2))
    return t


def _fgl_kernel(xa_ref, xb_ref, nf_ref, ft_ref, bias_ref, out_ref):
    nf = nf_ref[...]
    t = jnp.concatenate(
        [_grouped_max(xa_ref[...] * nf[None, :, :]),
         _grouped_max(xb_ref[...] * nf[None, :, :])],
        axis=0,
    )
    jj = jax.lax.broadcasted_iota(jnp.int32, (USED, OUTN), 0)
    oo = jax.lax.broadcasted_iota(jnp.int32, (USED, OUTN), 1)
    sel = (jj == oo * MAXD).astype(jnp.float32)
    m = jax.lax.dot_general(
        t, sel,
        dimension_numbers=(((2,), (0,)), ((), ())),
        preferred_element_type=jnp.float32,
    )
    ft = ft_ref[...]
    out = jax.lax.dot_general(
        m, ft,
        dimension_numbers=(((1,), (0,)), ((), ())),
        preferred_element_type=jnp.float32,
    )
    out = out.transpose(0, 2, 1)
    out_ref[...] = out + bias_ref[...][None, :, :]


def kernel(x, nf_weight, ft_weight, bias):
    grid = (NB // BLK,)
    return pl.pallas_call(
        _fgl_kernel,
        grid=grid,
        in_specs=[
            pl.BlockSpec((HALF, INC, USED), lambda j: (2 * j, 0, 0)),
            pl.BlockSpec((HALF, INC, USED), lambda j: (2 * j + 1, 0, 0)),
            pl.BlockSpec((INC, USED), lambda j: (0, 0)),
            pl.BlockSpec((INC, OUTC), lambda j: (0, 0)),
            pl.BlockSpec((OUTC, OUTN), lambda j: (0, 0)),
        ],
        out_specs=pl.BlockSpec((BLK, OUTC, OUTN), lambda j: (j, 0, 0)),
        out_shape=jax.ShapeDtypeStruct((NB, OUTC, OUTN), jnp.float32),
    )(x, x, nf_weight, ft_weight, bias)
